# baseline (speedup 1.0000x reference)
import functools

import jax
import jax.numpy as jnp
from jax import lax
from jax.experimental import pallas as pl
from jax.experimental.pallas import tpu as pltpu
from jax.experimental.pallas import tpu_sc as plsc

_EPS = 1e-5
_CHUNK = 512
_GRP = 16

_GDN = lax.GatherDimensionNumbers(
    offset_dims=(), collapsed_slice_dims=(0,), start_index_map=(0,))


def _lane_splat(vec, rsplat):
    return lax.gather(vec, rsplat[:, None], _GDN, (1,),
                      mode=lax.GatherScatterMode.PROMISE_IN_BOUNDS)


@functools.lru_cache(maxsize=None)
def _make_fused(vocab: int, num_rows: int, d_model: int, seq_len: int, chunk: int):
    info = plsc.get_sparse_core_info()
    nc, ns = info.num_cores, info.num_subcores
    nw = nc * ns
    rpw = num_rows // nw
    n_chunks = rpw // chunk
    n_groups = chunk // _GRP
    nv = d_model // 16
    assert num_rows % nw == 0 and rpw % chunk == 0 and chunk % _GRP == 0

    mesh = plsc.VectorSubcoreMesh(core_axis_name="c", subcore_axis_name="s")

    @functools.partial(
        pl.kernel,
        mesh=mesh,
        compiler_params=pltpu.CompilerParams(
            use_tc_tiling_on_sc=False, needs_layout_passes=False),
        out_type=jax.ShapeDtypeStruct((num_rows, d_model), jnp.float32),
        scratch_types=[
            pltpu.VMEM((chunk,), jnp.int32),
            pltpu.VMEM((chunk, d_model), jnp.float32),
            pltpu.VMEM((seq_len, d_model), jnp.float32),
            pltpu.VMEM((d_model,), jnp.float32),
            pltpu.VMEM((d_model,), jnp.float32),
            pltpu.VMEM((_GRP * 17,), jnp.float32),
            pltpu.VMEM((_GRP * 17,), jnp.float32),
            pltpu.SemaphoreType.DMA,
        ],
    )
    def fused_k(idx_hbm, table_hbm, pos_hbm, gamma_hbm, beta_hbm, out_hbm,
                idx_v, rows_v, pos_v, g_v, b_v, s_v, q_v, sem):
        wid = lax.axis_index("s") * nc + lax.axis_index("c")
        base = wid * rpw
        pltpu.sync_copy(pos_hbm, pos_v)
        pltpu.sync_copy(gamma_hbm, g_v)
        pltpu.sync_copy(beta_hbm, b_v)
        lanes = jnp.arange(16, dtype=jnp.int32)
        lanes17 = lanes * 17
        gk = [g_v[pl.ds(16 * k, 16)] for k in range(nv)]
        bk = [b_v[pl.ds(16 * k, 16)] for k in range(nv)]

        def chunk_body(g, carry):
            off = base + g * chunk
            pltpu.sync_copy(idx_hbm.at[pl.ds(off, chunk)], idx_v)
            pltpu.async_copy(table_hbm.at[idx_v], rows_v, sem).wait()
            p_chunk = lax.rem(off, seq_len)

            def group_body(t, c2):
                j0 = t * _GRP
                pg = lax.rem(p_chunk + j0, seq_len)
                for r in range(_GRP):
                    j = j0 + r
                    p = pg + r
                    p = lax.select(p >= seq_len, p - seq_len, p)
                    h = [rows_v[j, pl.ds(16 * k, 16)] + pos_v[p, pl.ds(16 * k, 16)]
                         for k in range(nv)]
                    for k in range(nv):
                        rows_v[j, pl.ds(16 * k, 16)] = h[k]
                    s = (h[0] + h[1]) + (h[2] + h[3])
                    q = (h[0] * h[0] + h[1] * h[1]) + (h[2] * h[2] + h[3] * h[3])
                    plsc.store_scatter(s_v, [lanes + (17 * r)], s)
                    plsc.store_scatter(q_v, [lanes + (17 * r)], q)
                ssum = plsc.load_gather(s_v, [lanes17])
                qsum = plsc.load_gather(q_v, [lanes17])
                for l in range(1, 16):
                    ssum = ssum + plsc.load_gather(s_v, [lanes17 + l])
                    qsum = qsum + plsc.load_gather(q_v, [lanes17 + l])
                inv_d = jnp.float32(1.0 / d_model)
                mean = ssum * inv_d
                var = qsum * inv_d - mean * mean + jnp.float32(_EPS)
                i32 = lax.bitcast_convert_type(var, jnp.int32)
                y = lax.bitcast_convert_type(
                    jnp.int32(0x5F3759DF) - lax.shift_right_arithmetic(i32, 1),
                    jnp.float32)
                for _ in range(3):
                    y = y * (jnp.float32(1.5) - jnp.float32(0.5) * var * y * y)
                for r in range(_GRP):
                    j = j0 + r
                    rsplat = jnp.full((16,), r, dtype=jnp.int32)
                    m = _lane_splat(mean, rsplat)
                    sd = _lane_splat(y, rsplat)
                    for k in range(nv):
                        hk = rows_v[j, pl.ds(16 * k, 16)]
                        rows_v[j, pl.ds(16 * k, 16)] = (hk - m) * sd * gk[k] + bk[k]
                return c2

            lax.fori_loop(0, n_groups, group_body, 0)
            pltpu.sync_copy(rows_v, out_hbm.at[pl.ds(off, chunk)])
            return carry

        lax.fori_loop(0, n_chunks, chunk_body, 0)

    return fused_k


def kernel(x, tok_table, pos_table, gamma, beta):
    bt, s = x.shape
    vocab, dm = tok_table.shape
    num_rows = bt * s
    idx = x.reshape(num_rows).astype(jnp.int32)
    fused = _make_fused(vocab, num_rows, dm, s, _CHUNK)
    out = fused(idx, tok_table, pos_table, gamma, beta)
    return out.reshape(bt, s, dm)


# speedup vs baseline: 1.0644x; 1.0644x over previous
import functools

import jax
import jax.numpy as jnp
from jax import lax
from jax.experimental import pallas as pl
from jax.experimental.pallas import tpu as pltpu
from jax.experimental.pallas import tpu_sc as plsc

_EPS = 1e-5
_CHUNK = 512
_GRP = 16

_GDN = lax.GatherDimensionNumbers(
    offset_dims=(), collapsed_slice_dims=(0,), start_index_map=(0,))


def _lane_splat(vec, rsplat):
    return lax.gather(vec, rsplat[:, None], _GDN, (1,),
                      mode=lax.GatherScatterMode.PROMISE_IN_BOUNDS)


@functools.lru_cache(maxsize=None)
def _make_fused(vocab: int, num_rows: int, d_model: int, seq_len: int, chunk: int):
    info = plsc.get_sparse_core_info()
    nc, ns = info.num_cores, info.num_subcores
    nw = nc * ns
    rpw = num_rows // nw
    n_chunks = rpw // chunk
    n_groups = chunk // _GRP
    nv = d_model // 16
    assert num_rows % nw == 0 and rpw % chunk == 0 and chunk % _GRP == 0

    mesh = plsc.VectorSubcoreMesh(core_axis_name="c", subcore_axis_name="s")

    @functools.partial(
        pl.kernel,
        mesh=mesh,
        compiler_params=pltpu.CompilerParams(
            use_tc_tiling_on_sc=False, needs_layout_passes=False),
        out_type=jax.ShapeDtypeStruct((num_rows, d_model), jnp.float32),
        scratch_types=[
            pltpu.VMEM((rpw,), jnp.int32),
            pltpu.VMEM((chunk, d_model), jnp.float32),
            pltpu.VMEM((chunk, d_model), jnp.float32),
            pltpu.VMEM((seq_len, d_model), jnp.float32),
            pltpu.VMEM((d_model,), jnp.float32),
            pltpu.VMEM((d_model,), jnp.float32),
            pltpu.VMEM((_GRP * 17,), jnp.float32),
            pltpu.VMEM((_GRP * 17,), jnp.float32),
            pltpu.SemaphoreType.DMA,
            pltpu.SemaphoreType.DMA,
        ],
    )
    def fused_k(idx_hbm, table_hbm, pos_hbm, gamma_hbm, beta_hbm, out_hbm,
                idx_v, rows_v0, rows_v1, pos_v, g_v, b_v, s_v, q_v, sem0, sem1):
        wid = lax.axis_index("s") * nc + lax.axis_index("c")
        base = wid * rpw
        pltpu.sync_copy(idx_hbm.at[pl.ds(base, rpw)], idx_v)
        pltpu.sync_copy(pos_hbm, pos_v)
        pltpu.sync_copy(gamma_hbm, g_v)
        pltpu.sync_copy(beta_hbm, b_v)
        lanes = jnp.arange(16, dtype=jnp.int32)
        lanes17 = lanes * 17
        gk = [g_v[pl.ds(16 * k, 16)] for k in range(nv)]
        bk = [b_v[pl.ds(16 * k, 16)] for k in range(nv)]
        bufs = ((rows_v0, sem0), (rows_v1, sem1))

        def fire(g, buf, sem):
            pltpu.async_copy(
                table_hbm.at[idx_v.at[pl.ds(g * chunk, chunk)]], buf, sem)

        def drain(buf, sem):
            pltpu.make_async_copy(
                table_hbm.at[idx_v.at[pl.ds(0, chunk)]], buf, sem).wait()

        def process(g, rows_v):
            off = base + g * chunk
            p_chunk = lax.rem(off, seq_len)

            def group_body(t, c2):
                j0 = t * _GRP
                pg = lax.rem(p_chunk + j0, seq_len)
                for r in range(_GRP):
                    j = j0 + r
                    p = pg + r
                    p = lax.select(p >= seq_len, p - seq_len, p)
                    h = [rows_v[j, pl.ds(16 * k, 16)] + pos_v[p, pl.ds(16 * k, 16)]
                         for k in range(nv)]
                    for k in range(nv):
                        rows_v[j, pl.ds(16 * k, 16)] = h[k]
                    s = (h[0] + h[1]) + (h[2] + h[3])
                    q = (h[0] * h[0] + h[1] * h[1]) + (h[2] * h[2] + h[3] * h[3])
                    plsc.store_scatter(s_v, [lanes + (17 * r)], s)
                    plsc.store_scatter(q_v, [lanes + (17 * r)], q)
                ssum = plsc.load_gather(s_v, [lanes17])
                qsum = plsc.load_gather(q_v, [lanes17])
                for l in range(1, 16):
                    ssum = ssum + plsc.load_gather(s_v, [lanes17 + l])
                    qsum = qsum + plsc.load_gather(q_v, [lanes17 + l])
                inv_d = jnp.float32(1.0 / d_model)
                mean = ssum * inv_d
                var = qsum * inv_d - mean * mean + jnp.float32(_EPS)
                i32 = lax.bitcast_convert_type(var, jnp.int32)
                y = lax.bitcast_convert_type(
                    jnp.int32(0x5F3759DF) - lax.shift_right_arithmetic(i32, 1),
                    jnp.float32)
                for _ in range(3):
                    y = y * (jnp.float32(1.5) - jnp.float32(0.5) * var * y * y)
                for r in range(_GRP):
                    j = j0 + r
                    rsplat = jnp.full((16,), r, dtype=jnp.int32)
                    m = _lane_splat(mean, rsplat)
                    sd = _lane_splat(y, rsplat)
                    for k in range(nv):
                        hk = rows_v[j, pl.ds(16 * k, 16)]
                        rows_v[j, pl.ds(16 * k, 16)] = (hk - m) * sd * gk[k] + bk[k]
                return c2

            lax.fori_loop(0, n_groups, group_body, 0)
            pltpu.sync_copy(rows_v, out_hbm.at[pl.ds(off, chunk)])

        fire(0, *bufs[0])

        def pair_body(gp, carry):
            g0 = gp * 2
            drain(*bufs[0])
            fire(g0 + 1, *bufs[1])
            process(g0, bufs[0][0])
            drain(*bufs[1])

            @pl.when(g0 + 2 < n_chunks)
            def _():
                fire(g0 + 2, *bufs[0])

            process(g0 + 1, bufs[1][0])
            return carry

        lax.fori_loop(0, n_chunks // 2, pair_body, 0)

    return fused_k


def kernel(x, tok_table, pos_table, gamma, beta):
    bt, s = x.shape
    vocab, dm = tok_table.shape
    num_rows = bt * s
    idx = x.reshape(num_rows).astype(jnp.int32)
    fused = _make_fused(vocab, num_rows, dm, s, _CHUNK)
    out = fused(idx, tok_table, pos_table, gamma, beta)
    return out.reshape(bt, s, dm)


# speedup vs baseline: 1.0657x; 1.0012x over previous
import functools

import jax
import jax.numpy as jnp
from jax import lax
from jax.experimental import pallas as pl
from jax.experimental.pallas import tpu as pltpu
from jax.experimental.pallas import tpu_sc as plsc

_EPS = 1e-5
_CHUNK = 640
_GRP = 16

_GDN = lax.GatherDimensionNumbers(
    offset_dims=(), collapsed_slice_dims=(0,), start_index_map=(0,))


def _lane_splat(vec, rsplat):
    return lax.gather(vec, rsplat[:, None], _GDN, (1,),
                      mode=lax.GatherScatterMode.PROMISE_IN_BOUNDS)


@functools.lru_cache(maxsize=None)
def _make_fused(vocab: int, num_rows: int, d_model: int, seq_len: int, chunk: int):
    info = plsc.get_sparse_core_info()
    nc, ns = info.num_cores, info.num_subcores
    nw = nc * ns
    rpw = num_rows // nw
    n_chunks = rpw // chunk
    n_groups = chunk // _GRP
    nv = d_model // 16
    assert num_rows % nw == 0 and rpw % chunk == 0 and chunk % _GRP == 0

    mesh = plsc.VectorSubcoreMesh(core_axis_name="c", subcore_axis_name="s")

    @functools.partial(
        pl.kernel,
        mesh=mesh,
        compiler_params=pltpu.CompilerParams(
            use_tc_tiling_on_sc=False, needs_layout_passes=False),
        out_type=jax.ShapeDtypeStruct((num_rows, d_model), jnp.float32),
        scratch_types=[
            pltpu.VMEM((rpw,), jnp.int32),
            pltpu.VMEM((chunk, d_model), jnp.float32),
            pltpu.VMEM((chunk, d_model), jnp.float32),
            pltpu.VMEM((seq_len, d_model), jnp.float32),
            pltpu.VMEM((d_model,), jnp.float32),
            pltpu.VMEM((d_model,), jnp.float32),
            pltpu.VMEM((_GRP * 17,), jnp.float32),
            pltpu.VMEM((_GRP * 17,), jnp.float32),
            pltpu.SemaphoreType.DMA,
            pltpu.SemaphoreType.DMA,
        ],
    )
    def fused_k(idx_hbm, table_hbm, pos_hbm, gamma_hbm, beta_hbm, out_hbm,
                idx_v, rows_v0, rows_v1, pos_v, g_v, b_v, s_v, q_v, sem0, sem1):
        wid = lax.axis_index("s") * nc + lax.axis_index("c")
        base = wid * rpw
        pltpu.sync_copy(idx_hbm.at[pl.ds(base, rpw)], idx_v)
        pltpu.sync_copy(pos_hbm, pos_v)
        pltpu.sync_copy(gamma_hbm, g_v)
        pltpu.sync_copy(beta_hbm, b_v)
        lanes = jnp.arange(16, dtype=jnp.int32)
        lanes17 = lanes * 17
        gk = [g_v[pl.ds(16 * k, 16)] for k in range(nv)]
        bk = [b_v[pl.ds(16 * k, 16)] for k in range(nv)]
        bufs = ((rows_v0, sem0), (rows_v1, sem1))

        def fire(g, buf, sem):
            pltpu.async_copy(
                table_hbm.at[idx_v.at[pl.ds(g * chunk, chunk)]], buf, sem)

        def drain(buf, sem):
            pltpu.make_async_copy(
                table_hbm.at[idx_v.at[pl.ds(0, chunk)]], buf, sem).wait()

        def process(g, rows_v):
            off = base + g * chunk
            p_chunk = lax.rem(off, seq_len)

            def group_body(t, c2):
                j0 = t * _GRP
                pg = lax.rem(p_chunk + j0, seq_len)
                for r in range(_GRP):
                    j = j0 + r
                    p = pg + r
                    p = lax.select(p >= seq_len, p - seq_len, p)
                    h = [rows_v[j, pl.ds(16 * k, 16)] + pos_v[p, pl.ds(16 * k, 16)]
                         for k in range(nv)]
                    for k in range(nv):
                        rows_v[j, pl.ds(16 * k, 16)] = h[k]
                    s = (h[0] + h[1]) + (h[2] + h[3])
                    q = (h[0] * h[0] + h[1] * h[1]) + (h[2] * h[2] + h[3] * h[3])
                    plsc.store_scatter(s_v, [lanes + (17 * r)], s)
                    plsc.store_scatter(q_v, [lanes + (17 * r)], q)
                ssum = plsc.load_gather(s_v, [lanes17])
                qsum = plsc.load_gather(q_v, [lanes17])
                for l in range(1, 16):
                    ssum = ssum + plsc.load_gather(s_v, [lanes17 + l])
                    qsum = qsum + plsc.load_gather(q_v, [lanes17 + l])
                inv_d = jnp.float32(1.0 / d_model)
                mean = ssum * inv_d
                var = qsum * inv_d - mean * mean + jnp.float32(_EPS)
                i32 = lax.bitcast_convert_type(var, jnp.int32)
                y = lax.bitcast_convert_type(
                    jnp.int32(0x5F3759DF) - lax.shift_right_arithmetic(i32, 1),
                    jnp.float32)
                for _ in range(3):
                    y = y * (jnp.float32(1.5) - jnp.float32(0.5) * var * y * y)
                for r in range(_GRP):
                    j = j0 + r
                    rsplat = jnp.full((16,), r, dtype=jnp.int32)
                    m = _lane_splat(mean, rsplat)
                    sd = _lane_splat(y, rsplat)
                    for k in range(nv):
                        hk = rows_v[j, pl.ds(16 * k, 16)]
                        rows_v[j, pl.ds(16 * k, 16)] = (hk - m) * sd * gk[k] + bk[k]
                return c2

            lax.fori_loop(0, n_groups, group_body, 0)
            pltpu.sync_copy(rows_v, out_hbm.at[pl.ds(off, chunk)])

        fire(0, *bufs[0])

        def pair_body(gp, carry):
            g0 = gp * 2
            drain(*bufs[0])
            fire(g0 + 1, *bufs[1])
            process(g0, bufs[0][0])
            drain(*bufs[1])

            @pl.when(g0 + 2 < n_chunks)
            def _():
                fire(g0 + 2, *bufs[0])

            process(g0 + 1, bufs[1][0])
            return carry

        lax.fori_loop(0, n_chunks // 2, pair_body, 0)

    return fused_k


def kernel(x, tok_table, pos_table, gamma, beta):
    bt, s = x.shape
    vocab, dm = tok_table.shape
    num_rows = bt * s
    idx = x.reshape(num_rows).astype(jnp.int32)
    fused = _make_fused(vocab, num_rows, dm, s, _CHUNK)
    out = fused(idx, tok_table, pos_table, gamma, beta)
    return out.reshape(bt, s, dm)
